# baseline (speedup 1.0000x reference)
import functools

import jax
import jax.numpy as jnp
from jax import lax
from jax.experimental import pallas as pl
from jax.experimental.pallas import tpu as pltpu
from jax.experimental.pallas import tpu_sc as plsc

B = 16384
EMBED_DIM = 16
NC = 2
NS = 16
NW = NC * NS
B_PER_W = B // NW


def _sc_gather_body(ugrp_hbm, igrp_hbm, utab_hbm, itab_hbm,
                    uout_hbm, iout_hbm,
                    idx_v, rows_v, sem):
    wid = lax.axis_index("s") * NC + lax.axis_index("c")
    base = wid * B_PER_W
    for grp_hbm, tab_hbm, out_hbm in ((ugrp_hbm, utab_hbm, uout_hbm),
                                      (igrp_hbm, itab_hbm, iout_hbm)):
        pltpu.sync_copy(grp_hbm.at[pl.ds(base, B_PER_W)], idx_v)
        pltpu.async_copy(tab_hbm.at[idx_v], rows_v, sem).wait()
        pltpu.sync_copy(rows_v, out_hbm.at[pl.ds(base, B_PER_W)])


@jax.jit
def _sc_gather(user_grp, item_grp, utab128, itab128):
    mesh = plsc.VectorSubcoreMesh(core_axis_name="c", subcore_axis_name="s")
    f = pl.kernel(
        _sc_gather_body,
        out_type=(
            jax.ShapeDtypeStruct((B, 128), jnp.float32),
            jax.ShapeDtypeStruct((B, 128), jnp.float32),
        ),
        mesh=mesh,
        scratch_types=[
            pltpu.VMEM((B_PER_W,), jnp.int32),
            pltpu.VMEM((B_PER_W, 128), jnp.float32),
            pltpu.SemaphoreType.DMA,
        ],
    )
    return f(user_grp, item_grp, utab128, itab128)


def _extract(grp, off):
    out = jnp.zeros((grp.shape[0], EMBED_DIM), jnp.float32)
    for s in range(8):
        piece = grp[:, s * EMBED_DIM:(s + 1) * EMBED_DIM]
        out = out + jnp.where(off == s, piece, 0.0)
    return out


def _mlp_body(ug_ref, ig_ref, uo_ref, io_ref, dc_ref, w1u_ref, w1i_ref,
              w1dc_ref, b1_ref, w2_ref, b2_ref, wh_ref, bh_ref, out_ref):
    u = _extract(ug_ref[...], uo_ref[...])
    i = _extract(ig_ref[...], io_ref[...])
    h = (jnp.dot(u, w1u_ref[...], preferred_element_type=jnp.float32)
         + jnp.dot(i, w1i_ref[...], preferred_element_type=jnp.float32)
         + jnp.dot(dc_ref[...], w1dc_ref[...], preferred_element_type=jnp.float32)
         + b1_ref[...])
    h = jnp.maximum(h, 0.0)
    h = jnp.dot(h, w2_ref[...], preferred_element_type=jnp.float32) + b2_ref[...]
    h = jnp.maximum(h, 0.0)
    out_ref[...] = jax.nn.sigmoid(
        jnp.dot(h, wh_ref[...], preferred_element_type=jnp.float32) + bh_ref[...])


@jax.jit
def _tc_mlp(u_grp, i_grp, u_off, i_off, dc, w1u, w1i, w1dc, b1, w2, b2, wh, bh):
    BB = 2048
    grid = (B // BB,)
    dcdim = dc.shape[1]
    return pl.pallas_call(
        _mlp_body,
        grid=grid,
        in_specs=[
            pl.BlockSpec((BB, 128), lambda i: (i, 0)),
            pl.BlockSpec((BB, 128), lambda i: (i, 0)),
            pl.BlockSpec((BB, 1), lambda i: (i, 0)),
            pl.BlockSpec((BB, 1), lambda i: (i, 0)),
            pl.BlockSpec((BB, dcdim), lambda i: (i, 0)),
            pl.BlockSpec(w1u.shape, lambda i: (0, 0)),
            pl.BlockSpec(w1i.shape, lambda i: (0, 0)),
            pl.BlockSpec(w1dc.shape, lambda i: (0, 0)),
            pl.BlockSpec(b1.shape, lambda i: (0, 0)),
            pl.BlockSpec(w2.shape, lambda i: (0, 0)),
            pl.BlockSpec(b2.shape, lambda i: (0, 0)),
            pl.BlockSpec(wh.shape, lambda i: (0, 0)),
            pl.BlockSpec(bh.shape, lambda i: (0, 0)),
        ],
        out_specs=pl.BlockSpec((BB, 2), lambda i: (i, 0)),
        out_shape=jax.ShapeDtypeStruct((B, 2), jnp.float32),
    )(u_grp, i_grp, u_off, i_off, dc, w1u, w1i, w1dc, b1, w2, b2, wh, bh)


def kernel(user_idx, item_idx, dense_feats, comment_emb, user_table, item_table,
           W1, b1, W2, b2, ctr_w, ctr_b, cvr_w, cvr_b):
    user_idx = user_idx.astype(jnp.int32)
    item_idx = item_idx.astype(jnp.int32)
    utab128 = user_table.reshape(-1, 128)
    itab128 = item_table.reshape(-1, 128)
    u_grp, i_grp = _sc_gather(user_idx >> 3, item_idx >> 3, utab128, itab128)
    u_off = (user_idx & 7)[:, None]
    i_off = (item_idx & 7)[:, None]
    dc = jnp.concatenate([dense_feats, comment_emb], axis=-1)
    w1u = W1[:EMBED_DIM]
    w1i = W1[EMBED_DIM:2 * EMBED_DIM]
    w1dc = W1[2 * EMBED_DIM:]
    wh = jnp.concatenate([ctr_w, cvr_w], axis=1)
    bh = jnp.stack([ctr_b[0], cvr_b[0]])[None, :]
    out = _tc_mlp(u_grp, i_grp, u_off, i_off, dc, w1u, w1i, w1dc, b1[None, :],
                  W2, b2[None, :], wh, bh)
    return out[:, 0], out[:, 1]


# speedup vs baseline: 4.6351x; 4.6351x over previous
import functools

import jax
import jax.numpy as jnp
from jax import lax
from jax.experimental import pallas as pl
from jax.experimental.pallas import tpu as pltpu
from jax.experimental.pallas import tpu_sc as plsc

B = 16384
EMBED_DIM = 16
NV = 1000000
NC = 2
NS = 16
NW = NC * NS
B_PER_W = B // NW
KBUF = 16


def _sc_gather_body(uidx_hbm, iidx_hbm, utabT_hbm, itabT_hbm,
                    uout_hbm, iout_hbm,
                    idx_v, ring, rows_v, sems):
    wid = lax.axis_index("s") * NC + lax.axis_index("c")
    base = wid * B_PER_W
    lane = lax.iota(jnp.int32, EMBED_DIM)

    n_waves = B_PER_W // 16

    def gather_one(idx_hbm, tabT_hbm, out_hbm):
        pltpu.sync_copy(idx_hbm.at[pl.ds(base, B_PER_W)], idx_v)

        def fire(i, slot):
            c0 = pl.multiple_of((i // 128) * 128, 128)
            pltpu.async_copy(tabT_hbm.at[:, pl.ds(c0, 128)],
                             ring.at[slot], sems.at[slot])

        iv0 = idx_v[pl.ds(0, 16)]
        for k in range(16):
            fire(iv0[k], k)

        def body(q, carry):
            iv = idx_v[pl.ds(q * 16, 16)]
            qn = jnp.minimum(q + 1, n_waves - 1)
            ivn = idx_v[pl.ds(qn * 16, 16)]
            for k in range(16):
                pltpu.make_async_copy(
                    tabT_hbm.at[:, pl.ds(0, 128)], ring.at[k], sems.at[k]
                ).wait()
                di = lax.rem(iv[k], 128)
                vals = plsc.load_gather(ring.at[k], [lane, di + 0 * lane])
                plsc.store_scatter(
                    rows_v,
                    [jnp.full((EMBED_DIM,), 0, jnp.int32) + (q * 16 + k), lane],
                    vals)

                @pl.when(q + 1 < n_waves)
                def _refill():
                    fire(ivn[k], k)

            return carry

        lax.fori_loop(0, n_waves, body, 0)
        pltpu.sync_copy(rows_v, out_hbm.at[pl.ds(base, B_PER_W)])

    gather_one(uidx_hbm, utabT_hbm, uout_hbm)
    gather_one(iidx_hbm, itabT_hbm, iout_hbm)


@jax.jit
def _sc_gather(user_idx, item_idx, utabT, itabT):
    mesh = plsc.VectorSubcoreMesh(core_axis_name="c", subcore_axis_name="s")
    f = pl.kernel(
        _sc_gather_body,
        out_type=(
            jax.ShapeDtypeStruct((B, EMBED_DIM), jnp.float32),
            jax.ShapeDtypeStruct((B, EMBED_DIM), jnp.float32),
        ),
        mesh=mesh,
        scratch_types=[
            pltpu.VMEM((B_PER_W,), jnp.int32),
            pltpu.VMEM((KBUF, EMBED_DIM, 128), jnp.float32),
            pltpu.VMEM((B_PER_W, EMBED_DIM), jnp.float32),
            pltpu.SemaphoreType.DMA((KBUF,)),
        ],
        compiler_params=pltpu.CompilerParams(needs_layout_passes=False),
    )
    return f(user_idx, item_idx, utabT, itabT)


def _mlp_body(u_ref, i_ref, dc_ref, w1u_ref, w1i_ref, w1dc_ref, b1_ref,
              w2_ref, b2_ref, wh_ref, bh_ref, out_ref):
    h = (jnp.dot(u_ref[...], w1u_ref[...], preferred_element_type=jnp.float32)
         + jnp.dot(i_ref[...], w1i_ref[...], preferred_element_type=jnp.float32)
         + jnp.dot(dc_ref[...], w1dc_ref[...], preferred_element_type=jnp.float32)
         + b1_ref[...])
    h = jnp.maximum(h, 0.0)
    h = jnp.dot(h, w2_ref[...], preferred_element_type=jnp.float32) + b2_ref[...]
    h = jnp.maximum(h, 0.0)
    out_ref[...] = jax.nn.sigmoid(
        jnp.dot(h, wh_ref[...], preferred_element_type=jnp.float32) + bh_ref[...])


@jax.jit
def _tc_mlp(u_emb, i_emb, dc, w1u, w1i, w1dc, b1, w2, b2, wh, bh):
    BB = 2048
    grid = (B // BB,)
    dcdim = dc.shape[1]
    return pl.pallas_call(
        _mlp_body,
        grid=grid,
        in_specs=[
            pl.BlockSpec((BB, EMBED_DIM), lambda i: (i, 0)),
            pl.BlockSpec((BB, EMBED_DIM), lambda i: (i, 0)),
            pl.BlockSpec((BB, dcdim), lambda i: (i, 0)),
            pl.BlockSpec(w1u.shape, lambda i: (0, 0)),
            pl.BlockSpec(w1i.shape, lambda i: (0, 0)),
            pl.BlockSpec(w1dc.shape, lambda i: (0, 0)),
            pl.BlockSpec(b1.shape, lambda i: (0, 0)),
            pl.BlockSpec(w2.shape, lambda i: (0, 0)),
            pl.BlockSpec(b2.shape, lambda i: (0, 0)),
            pl.BlockSpec(wh.shape, lambda i: (0, 0)),
            pl.BlockSpec(bh.shape, lambda i: (0, 0)),
        ],
        out_specs=pl.BlockSpec((BB, 2), lambda i: (i, 0)),
        out_shape=jax.ShapeDtypeStruct((B, 2), jnp.float32),
    )(u_emb, i_emb, dc, w1u, w1i, w1dc, b1, w2, b2, wh, bh)


def kernel(user_idx, item_idx, dense_feats, comment_emb, user_table, item_table,
           W1, b1, W2, b2, ctr_w, ctr_b, cvr_w, cvr_b):
    user_idx = user_idx.astype(jnp.int32)
    item_idx = item_idx.astype(jnp.int32)
    u_emb, i_emb = _sc_gather(user_idx, item_idx, user_table.T, item_table.T)
    dc = jnp.concatenate([dense_feats, comment_emb], axis=-1)
    w1u = W1[:EMBED_DIM]
    w1i = W1[EMBED_DIM:2 * EMBED_DIM]
    w1dc = W1[2 * EMBED_DIM:]
    wh = jnp.concatenate([ctr_w, cvr_w], axis=1)
    bh = jnp.stack([ctr_b[0], cvr_b[0]])[None, :]
    out = _tc_mlp(u_emb, i_emb, dc, w1u, w1i, w1dc, b1[None, :], W2,
                  b2[None, :], wh, bh)
    return out[:, 0], out[:, 1]
